# Initial kernel scaffold; baseline (speedup 1.0000x reference)
import jax
import jax.numpy as jnp
from jax.experimental import pallas as pl


def kernel(coords, feats, W1, W2, W3, W4, W5, Wf1, bf1, Wf2, bf2, Wf3, bf3, Wf4, bf4, Wsem, bsem):
    raise NotImplementedError("write your pallas kernel here")


# speedup vs baseline: 1.0118x; 1.0118x over previous
import functools

import jax
import jax.numpy as jnp
from jax import lax
from jax.experimental import pallas as pl
from jax.experimental.pallas import tpu as pltpu

K = 20


def _leaky(x):
    return jnp.where(x >= 0, x, 0.2 * x)


def _edge_layer_xla(x, W):
    inner = -2.0 * jnp.einsum('bnc,bmc->bnm', x, x)
    xx = jnp.sum(x ** 2, axis=-1)
    pd = -xx[:, :, None] - inner - xx[:, None, :]
    idx = lax.top_k(pd, K)[1]
    C = x.shape[-1]
    Wa, Wb = W[:, :C], W[:, C:]
    nbr = jax.vmap(lambda a, i: a[i])(x, idx)
    center = x[:, :, None, :]
    d16 = (nbr - center).astype(jnp.bfloat16)
    x16 = x.astype(jnp.bfloat16)
    alpha = jnp.einsum('bnkc,oc->bnko', d16, Wa.astype(jnp.bfloat16),
                       preferred_element_type=jnp.float32)
    beta = jnp.einsum('bnc,oc->bno', x16, Wb.astype(jnp.bfloat16),
                      preferred_element_type=jnp.float32)
    return _leaky(jnp.max(alpha, axis=2) + beta)


def _head_body(cat_ref, w5_ref, wf1_ref, bf1_ref, wf2_ref, bf2_ref,
               wf3_ref, bf3_ref, wf4_ref, bf4_ref, wsem_ref, bsem_ref,
               f1_ref, f2_ref, f3_ref, f4_ref, sem_ref):
    x = cat_ref[...]
    dn = (((1,), (1,)), ((), ()))
    h = _leaky(lax.dot_general(x, w5_ref[...], dn,
                               preferred_element_type=jnp.float32))
    f1 = lax.dot_general(h, wf1_ref[...], dn,
                         preferred_element_type=jnp.float32) + bf1_ref[...]
    f2 = lax.dot_general(h, wf2_ref[...], dn,
                         preferred_element_type=jnp.float32) + bf2_ref[...]
    f3 = lax.dot_general(h, wf3_ref[...], dn,
                         preferred_element_type=jnp.float32) + bf3_ref[...]
    f4 = lax.dot_general(h, wf4_ref[...], dn,
                         preferred_element_type=jnp.float32) + bf4_ref[...]
    sem = lax.dot_general(f4, wsem_ref[...], dn,
                          preferred_element_type=jnp.float32) + bsem_ref[...]
    f1_ref[...] = f1
    f2_ref[...] = f2
    f3_ref[...] = f3
    f4_ref[...] = f4
    sem_ref[...] = sem


@functools.partial(jax.jit, static_argnames=())
def _head(cat, W5, Wf1, bf1, Wf2, bf2, Wf3, bf3, Wf4, bf4, Wsem, bsem):
    B, N, C = cat.shape
    M = B * N
    TR = 512
    cat2 = cat.reshape(M, C)
    grid = (M // TR,)
    full = lambda s: pl.BlockSpec(s, lambda i: tuple(0 for _ in s))
    row = lambda c: pl.BlockSpec((TR, c), lambda i: (i, 0))
    out_shapes = [
        jax.ShapeDtypeStruct((M, 64), jnp.float32),
        jax.ShapeDtypeStruct((M, 128), jnp.float32),
        jax.ShapeDtypeStruct((M, 256), jnp.float32),
        jax.ShapeDtypeStruct((M, 512), jnp.float32),
        jax.ShapeDtypeStruct((M, 32), jnp.float32),
    ]
    wsem_p = jnp.zeros((32, 512), jnp.float32).at[:20].set(Wsem)
    bsem_p = jnp.zeros((32,), jnp.float32).at[:20].set(bsem)
    f1, f2, f3, f4, semp = pl.pallas_call(
        _head_body,
        grid=grid,
        in_specs=[
            row(512),
            full((512, 512)),
            full((64, 512)), full((64,)),
            full((128, 512)), full((128,)),
            full((256, 512)), full((256,)),
            full((512, 512)), full((512,)),
            full((32, 512)), full((32,)),
        ],
        out_specs=[row(64), row(128), row(256), row(512), row(32)],
        out_shape=out_shapes,
    )(cat2, W5, Wf1, bf1, Wf2, bf2, Wf3, bf3, Wf4, bf4, wsem_p, bsem_p)
    return (f1.reshape(B, N, 64), f2.reshape(B, N, 128),
            f3.reshape(B, N, 256), f4.reshape(B, N, 512),
            semp[:, :20].reshape(B, N, 20))


def kernel(coords, feats, W1, W2, W3, W4, W5, Wf1, bf1, Wf2, bf2, Wf3, bf3,
           Wf4, bf4, Wsem, bsem):
    B, N, _ = feats.shape
    x1 = _edge_layer_xla(feats, W1)
    x2 = _edge_layer_xla(x1, W2)
    x3 = _edge_layer_xla(x2, W3)
    x4 = _edge_layer_xla(x3, W4)
    cat = jnp.concatenate([x1, x2, x3, x4], axis=-1)
    f1, f2, f3, f4, sem = _head(cat, W5, Wf1, bf1, Wf2, bf2, Wf3, bf3,
                                Wf4, bf4, Wsem, bsem)
    masks = jnp.zeros((B, N), dtype=bool)
    return (f1, f2, f3, f4, coords, masks, sem)


# speedup vs baseline: 9.4550x; 9.3443x over previous
import functools

import jax
import jax.numpy as jnp
from jax import lax
from jax.experimental import pallas as pl
from jax.experimental.pallas import tpu as pltpu
from jax.experimental.pallas import tpu_sc as plsc

_K = 20
_KPAD = 32
_N = 2048
_B = 2
_M = _B * _N
_TOPK_L = 256
_TP = 512


def _leaky(x):
    return jnp.where(x >= 0, x, 0.2 * x)


def _pd_topk_body(xf_ref, xb_ref, idx_ref):
    b = pl.program_id(0)
    xm = xf_ref[0]
    xn = xb_ref[0]
    xm16 = xm.astype(jnp.bfloat16)
    xn16 = xn.astype(jnp.bfloat16)
    dn = (((1,), (1,)), ((), ()))
    inner = lax.dot_general(xm16, xn16, dn,
                            preferred_element_type=jnp.float32)
    xxm = jnp.sum(xm * xm, axis=1, keepdims=True)
    pdv = 2.0 * inner - xxm
    iota = lax.broadcasted_iota(jnp.int32, (_N, _TOPK_L), 0)
    base = b * _N
    for t in range(_K):
        gmax = jnp.max(pdv, axis=0, keepdims=True)
        hit = pdv == gmax
        idx_t = jnp.min(jnp.where(hit, iota, _N), axis=0,
                        keepdims=True)
        pdv = jnp.where(iota == idx_t, -jnp.inf, pdv)
        idx_ref[0, pl.ds(t, 1), :] = idx_t + base
    zero = jnp.zeros((1, _TOPK_L), jnp.int32)
    for t in range(_K, _KPAD):
        idx_ref[0, pl.ds(t, 1), :] = zero


def _pd_topk(x):
    B, N, C = x.shape
    idx = pl.pallas_call(
        _pd_topk_body,
        grid=(B, N // _TOPK_L),
        in_specs=[
            pl.BlockSpec((1, N, C), lambda b, j: (b, 0, 0)),
            pl.BlockSpec((1, _TOPK_L, C), lambda b, j: (b, j, 0)),
        ],
        out_specs=pl.BlockSpec((1, _KPAD, _TOPK_L), lambda b, j: (b, 0, j)),
        out_shape=jax.ShapeDtypeStruct((B, _KPAD, N), jnp.int32),
    )(x, x)
    return idx.reshape(B * _KPAD, N)


def _sc_gather(x_flat, idx2):
    M, C = x_flat.shape
    ppw = 128
    mesh = plsc.VectorSubcoreMesh(core_axis_name="c", subcore_axis_name="s")

    @functools.partial(
        pl.kernel, mesh=mesh,
        out_type=jax.ShapeDtypeStruct((_K, M, C), jnp.float32),
        scratch_types=[
            pltpu.VMEM((ppw,), jnp.int32),
            pltpu.VMEM((ppw, C), jnp.float32),
            pltpu.SemaphoreType.DMA,
        ],
    )
    def kern(x_hbm, idx_hbm, out_hbm, idx_v, rows_v, sem):
        cid = lax.axis_index("c")
        sid = lax.axis_index("s")
        wid = sid * 2 + cid
        b = wid // 16
        j = wid % 16
        for kk in range(_K):
            pltpu.sync_copy(idx_hbm.at[b * _KPAD + kk, pl.ds(j * ppw, ppw)],
                            idx_v)
            pltpu.async_copy(x_hbm.at[idx_v], rows_v, sem).wait()
            pltpu.sync_copy(rows_v,
                            out_hbm.at[kk, pl.ds(b * _N + j * ppw, ppw)])

    return kern(x_flat, idx2)


def _edge_body(x_ref, nbr_ref, wa_ref, wb_ref, o_ref):
    x = x_ref[...]
    x16 = x.astype(jnp.bfloat16)
    dn = (((1,), (1,)), ((), ()))
    beta = lax.dot_general(x16, wb_ref[...], dn,
                           preferred_element_type=jnp.float32)
    acc = None
    for kk in range(_K):
        d16 = (nbr_ref[kk] - x).astype(jnp.bfloat16)
        a = lax.dot_general(d16, wa_ref[...], dn,
                            preferred_element_type=jnp.float32)
        acc = a if acc is None else jnp.maximum(acc, a)
    o_ref[...] = _leaky(acc + beta)


def _edge_consume(x_flat, nbr, wa16, wb16):
    M, C = x_flat.shape
    out = wa16.shape[0]
    return pl.pallas_call(
        _edge_body,
        grid=(M // _TP,),
        in_specs=[
            pl.BlockSpec((_TP, C), lambda i: (i, 0)),
            pl.BlockSpec((_K, _TP, C), lambda i: (0, i, 0)),
            pl.BlockSpec((out, C), lambda i: (0, 0)),
            pl.BlockSpec((out, C), lambda i: (0, 0)),
        ],
        out_specs=pl.BlockSpec((_TP, out), lambda i: (i, 0)),
        out_shape=jax.ShapeDtypeStruct((M, out), jnp.float32),
    )(x_flat, nbr, wa16, wb16)


_CP = 128


def _edge_layer(x_bnc, W):
    B, N, C = x_bnc.shape
    Wa, Wb = W[:, :C], W[:, C:]
    if C != _CP:
        x_bnc = jnp.concatenate(
            [x_bnc, jnp.zeros((B, N, _CP - C), jnp.float32)], axis=-1)
        pad_w = jnp.zeros((W.shape[0], _CP - C), jnp.float32)
        Wa = jnp.concatenate([Wa, pad_w], axis=-1)
        Wb = jnp.concatenate([Wb, pad_w], axis=-1)
    idx2 = _pd_topk(x_bnc)
    x_flat = x_bnc.reshape(B * N, _CP)
    nbr = _sc_gather(x_flat, idx2)
    return _edge_consume(x_flat, nbr, Wa.astype(jnp.bfloat16),
                         Wb.astype(jnp.bfloat16))


def _head_body(x1_ref, x2_ref, x3_ref, x4_ref, w5a_ref, w5b_ref, w5c_ref,
               w5d_ref, wf1_ref, bf1_ref, wf2_ref, bf2_ref,
               wf3_ref, bf3_ref, wf4_ref, bf4_ref, wsem_ref, bsem_ref,
               f1_ref, f2_ref, f3_ref, f4_ref, sem_ref):
    dn = (((1,), (1,)), ((), ()))
    mm = lambda a, w: lax.dot_general(a, w, dn,
                                      preferred_element_type=jnp.float32)
    h = _leaky(mm(x1_ref[...], w5a_ref[...]) + mm(x2_ref[...], w5b_ref[...])
               + mm(x3_ref[...], w5c_ref[...]) + mm(x4_ref[...], w5d_ref[...]))
    f1 = mm(h, wf1_ref[...]) + bf1_ref[...]
    f2 = mm(h, wf2_ref[...]) + bf2_ref[...]
    f3 = mm(h, wf3_ref[...]) + bf3_ref[...]
    f4 = mm(h, wf4_ref[...]) + bf4_ref[...]
    sem = mm(f4, wsem_ref[...]) + bsem_ref[...]
    f1_ref[...] = f1
    f2_ref[...] = f2
    f3_ref[...] = f3
    f4_ref[...] = f4
    sem_ref[...] = sem


def _head(x1, x2, x3, x4, W5, Wf1, bf1, Wf2, bf2, Wf3, bf3, Wf4, bf4,
          Wsem, bsem):
    M = x1.shape[0]
    grid = (M // _TP,)
    full = lambda s: pl.BlockSpec(s, lambda i: tuple(0 for _ in s))
    row = lambda c: pl.BlockSpec((_TP, c), lambda i: (i, 0))
    out_shapes = [
        jax.ShapeDtypeStruct((M, 64), jnp.float32),
        jax.ShapeDtypeStruct((M, 128), jnp.float32),
        jax.ShapeDtypeStruct((M, 256), jnp.float32),
        jax.ShapeDtypeStruct((M, 512), jnp.float32),
        jax.ShapeDtypeStruct((M, 32), jnp.float32),
    ]
    wsem_p = jnp.zeros((32, 512), jnp.float32).at[:20].set(Wsem)
    bsem_p = jnp.zeros((32,), jnp.float32).at[:20].set(bsem)
    return pl.pallas_call(
        _head_body,
        grid=grid,
        in_specs=[
            row(64), row(64), row(128), row(256),
            full((512, 64)), full((512, 64)), full((512, 128)),
            full((512, 256)),
            full((64, 512)), full((64,)),
            full((128, 512)), full((128,)),
            full((256, 512)), full((256,)),
            full((512, 512)), full((512,)),
            full((32, 512)), full((32,)),
        ],
        out_specs=[row(64), row(128), row(256), row(512), row(32)],
        out_shape=out_shapes,
    )(x1, x2, x3, x4, W5[:, :64], W5[:, 64:128], W5[:, 128:256],
      W5[:, 256:512], Wf1, bf1, Wf2, bf2, Wf3, bf3, Wf4, bf4, wsem_p, bsem_p)


def kernel(coords, feats, W1, W2, W3, W4, W5, Wf1, bf1, Wf2, bf2, Wf3, bf3,
           Wf4, bf4, Wsem, bsem):
    B, N, _ = feats.shape
    x1 = _edge_layer(feats, W1)
    x2 = _edge_layer(x1.reshape(B, N, 64), W2)
    x3 = _edge_layer(x2.reshape(B, N, 64), W3)
    x4 = _edge_layer(x3.reshape(B, N, 128), W4)
    f1, f2, f3, f4, semp = _head(x1, x2, x3, x4, W5, Wf1, bf1, Wf2, bf2,
                                 Wf3, bf3, Wf4, bf4, Wsem, bsem)
    masks = jnp.zeros((B, N), dtype=bool)
    return (f1.reshape(B, N, 64), f2.reshape(B, N, 128),
            f3.reshape(B, N, 256), f4.reshape(B, N, 512), coords, masks,
            semp[:, :20].reshape(B, N, 20))
